# Initial kernel scaffold; baseline (speedup 1.0000x reference)
import jax
import jax.numpy as jnp
from jax.experimental import pallas as pl


def kernel(x_user, x_book, edge_index_ub, edge_index_bu, Wu, bu, Wb, bb, Wl0, Wr0, bl0, Wl1, Wr1, bl1, Wl2, Wr2, bl2, gu, betau, gb, betab):
    raise NotImplementedError("write your pallas kernel here")


# speedup vs baseline: 3.8626x; 3.8626x over previous
import functools

import jax
import jax.numpy as jnp
from jax import lax
from jax.experimental import pallas as pl
from jax.experimental.pallas import tpu as pltpu
from jax.experimental.pallas import tpu_sc as plsc

N = 10000
E = 320000
H = 128
C = 128
NT = 16
CPT = 160
NCH = CPT * NT
EPAD = NCH * C
NP = 10240
DUMMY = 10008
ROWS_PT = NP // NT
R = 1024
NB = NP // R


GRP = 8


def _agg_body(x2, gidx, sidx, zrows, orows, out, gixr, sixr, r0, r1, acc,
              sem0, sem1, *, with_gather):
  c = lax.axis_index("c")
  s = lax.axis_index("s")
  base = s * CPT

  if not with_gather:
    pltpu.sync_copy(orows, r0)
  pltpu.sync_copy(zrows, acc.at[pl.ds(s * ROWS_PT, ROWS_PT)])
  plsc.subcore_barrier()

  xc = x2.at[c]

  def group(g, _):
    gb = base + g * GRP
    pltpu.sync_copy(sidx.at[c].at[pl.ds(gb, GRP)], sixr)
    if with_gather:
      pltpu.sync_copy(gidx.at[c].at[pl.ds(gb, GRP)], gixr)
      pltpu.async_copy(xc.at[gixr.at[0]], r0, sem0)
      pltpu.async_copy(xc.at[gixr.at[1]], r1, sem1)
      for b in range(GRP):
        rb, sb = (r0, sem0) if b % 2 == 0 else (r1, sem1)
        pltpu.make_async_copy(xc.at[gixr.at[b]], rb, sb).wait()
        pltpu.sync_copy(rb, acc.at[sixr.at[b]], add=True)
        if b + 2 < GRP:
          pltpu.async_copy(xc.at[gixr.at[b + 2]], rb, sb)
    else:
      for b in range(GRP):
        pltpu.sync_copy(r0, acc.at[sixr.at[b]], add=True)
    return 0

  lax.fori_loop(0, CPT // GRP, group, 0)

  plsc.subcore_barrier()
  pltpu.sync_copy(acc.at[pl.ds(s * ROWS_PT, ROWS_PT)],
                  out.at[c].at[pl.ds(s * ROWS_PT, ROWS_PT)])


def _make_agg(with_gather):
  mesh = plsc.VectorSubcoreMesh(core_axis_name="c", subcore_axis_name="s")
  return functools.partial(
      pl.kernel,
      out_type=jax.ShapeDtypeStruct((2, NP, H), jnp.float32),
      mesh=mesh,
      scratch_types=[
          pltpu.VMEM((GRP, C), jnp.int32),
          pltpu.VMEM((GRP, C), jnp.int32),
          pltpu.VMEM((C, H), jnp.float32),
          pltpu.VMEM((C, H), jnp.float32),
          pltpu.VMEM_SHARED((NP, H), jnp.float32),
          pltpu.SemaphoreType.DMA,
          pltpu.SemaphoreType.DMA,
      ],
  )(functools.partial(_agg_body, with_gather=with_gather))


_agg_sum = _make_agg(True)
_agg_cnt = _make_agg(False)

_DN = (((1,), (0,)), ((), ()))


def _proj_body(xu_ref, xb_ref, w_ref, b_ref, out_ref):
  d = pl.program_id(0)
  x = jnp.where(d == 0, xu_ref[...], xb_ref[...])
  y = lax.dot_general(x, w_ref[0], _DN, precision=lax.Precision.HIGHEST)
  out_ref[0] = y + b_ref[0]


def _proj(xu, xb, wst, bst):
  return pl.pallas_call(
      _proj_body,
      grid=(2, NB),
      in_specs=[
          pl.BlockSpec((R, H), lambda d, i: (i, 0)),
          pl.BlockSpec((R, H), lambda d, i: (i, 0)),
          pl.BlockSpec((1, H, H), lambda d, i: (d, 0, 0)),
          pl.BlockSpec((1, 1, H), lambda d, i: (d, 0, 0)),
      ],
      out_specs=pl.BlockSpec((1, R, H), lambda d, i: (d, i, 0)),
      out_shape=jax.ShapeDtypeStruct((2, NP, H), jnp.float32),
  )(xu, xb, wst, bst)


def _dense_body(sum_ref, cnt_ref, x_ref, wl_ref, wr_ref, bl_ref, g_ref,
                b2_ref, out_ref, *, ln):
  sums = sum_ref[0]
  cnt = jnp.maximum(cnt_ref[0][:, :1], 1.0)
  agg = sums / cnt
  x = x_ref[0]
  y = lax.dot_general(agg, wl_ref[0], _DN, precision=lax.Precision.HIGHEST)
  y = y + lax.dot_general(x, wr_ref[0], _DN, precision=lax.Precision.HIGHEST)
  y = jnp.maximum(y + bl_ref[0], 0.0)
  if ln:
    m = jnp.mean(y, axis=1, keepdims=True)
    v = jnp.mean((y - m) * (y - m), axis=1, keepdims=True)
    y = (y - m) * lax.rsqrt(v + 1e-5) * g_ref[0] + b2_ref[0]
  out_ref[0] = y


def _dense(ln):
  return pl.pallas_call(
      functools.partial(_dense_body, ln=ln),
      grid=(2, NB),
      in_specs=[
          pl.BlockSpec((1, R, H), lambda d, i: (1 - d, i, 0)),
          pl.BlockSpec((1, R, H), lambda d, i: (1 - d, i, 0)),
          pl.BlockSpec((1, R, H), lambda d, i: (d, i, 0)),
          pl.BlockSpec((1, H, H), lambda d, i: (0, 0, 0)),
          pl.BlockSpec((1, H, H), lambda d, i: (0, 0, 0)),
          pl.BlockSpec((1, 1, H), lambda d, i: (0, 0, 0)),
          pl.BlockSpec((1, 1, H), lambda d, i: (d, 0, 0)),
          pl.BlockSpec((1, 1, H), lambda d, i: (d, 0, 0)),
      ],
      out_specs=pl.BlockSpec((1, R, H), lambda d, i: (d, i, 0)),
      out_shape=jax.ShapeDtypeStruct((2, NP, H), jnp.float32),
  )


_dense_mid = _dense(False)
_dense_last = _dense(True)


def _prep_idx(a):
  pad = jnp.full((EPAD - E,), DUMMY, jnp.int32)
  return jnp.concatenate([a.astype(jnp.int32), pad]).reshape(NCH, C)


def kernel(x_user, x_book, edge_index_ub, edge_index_bu, Wu, bu, Wb, bb,
           Wl0, Wr0, bl0, Wl1, Wr1, bl1, Wl2, Wr2, bl2, gu, betau, gb, betab):
  f32 = jnp.float32
  gidx = jnp.stack([_prep_idx(edge_index_ub[0]), _prep_idx(edge_index_bu[0])])
  sidx = jnp.stack([_prep_idx(edge_index_ub[1]), _prep_idx(edge_index_bu[1])])
  zrows = jnp.zeros((ROWS_PT, H), f32)
  orows = jnp.ones((C, H), f32)

  wst = jnp.stack([Wu, Wb])
  bst = jnp.stack([bu, bb])[:, None, :]
  g2 = jnp.stack([gu, gb])[:, None, :]
  beta2 = jnp.stack([betau, betab])[:, None, :]

  x2 = _proj(jnp.pad(x_user, ((0, NP - N), (0, 0))),
             jnp.pad(x_book, ((0, NP - N), (0, 0))), wst, bst)
  cnts = _agg_cnt(x2, gidx, sidx, zrows, orows)

  layers = [(Wl0, Wr0, bl0, False), (Wl1, Wr1, bl1, False),
            (Wl2, Wr2, bl2, True)]
  for (wl, wr, bl, last) in layers:
    sums = _agg_sum(x2, gidx, sidx, zrows, orows)
    dense = _dense_last if last else _dense_mid
    x2 = dense(sums, cnts, x2, wl[None], wr[None], bl[None, None], g2, beta2)

  return (x2[0, :N], x2[1, :N])


# speedup vs baseline: 3.8684x; 1.0015x over previous
import functools

import jax
import jax.numpy as jnp
from jax import lax
from jax.experimental import pallas as pl
from jax.experimental.pallas import tpu as pltpu
from jax.experimental.pallas import tpu_sc as plsc

N = 10000
E = 320000
H = 128
C = 128
NT = 16
CPT = 160
NCH = CPT * NT
EPAD = NCH * C
NP = 10240
DUMMY = 10008
ROWS_PT = NP // NT
R = 1024
NB = NP // R


GRP = 16
NG = CPT // GRP
PAD_ROW = NCH - 8


def _agg_body(x2, gidx, sidx, zrows, orows, out, six0, gix0, six1, gix1,
              r0, r1, acc, gsa, gsb, ssa, ssb, isem, *, with_gather):
  c = lax.axis_index("c")
  s = lax.axis_index("s")
  base = s * CPT
  xc = x2.at[c]
  sc = sidx.at[c]
  gc = gidx.at[c]

  if not with_gather:
    pltpu.sync_copy(orows, r0)
    pltpu.sync_copy(zrows, acc.at[pl.ds(s * ROWS_PT, ROWS_PT)])
    plsc.subcore_barrier()

    def group(g, _):
      pltpu.sync_copy(sc.at[pl.ds(base + g * GRP, GRP)], six0)
      for b in range(GRP):
        pltpu.sync_copy(r0, acc.at[six0.at[b]], add=True)
      return 0

    lax.fori_loop(0, NG, group, 0)
  else:
    pltpu.async_copy(sc.at[pl.ds(base, GRP)], six0, isem)
    pltpu.async_copy(gc.at[pl.ds(base, GRP)], gix0, isem)
    pltpu.sync_copy(zrows, acc.at[pl.ds(s * ROWS_PT, ROWS_PT)])
    plsc.subcore_barrier()

    rings = ((six0, gix0), (six1, gix1))

    def pair(p, _):
      for qg in (0, 1):
        g = p * 2 + qg
        sixq, gixq = rings[qg]
        sixn, gixn = rings[1 - qg]
        pltpu.make_async_copy(sc.at[pl.ds(base, GRP)], sixq, isem).wait()
        pltpu.make_async_copy(gc.at[pl.ds(base, GRP)], gixq, isem).wait()
        gnxt = base + jnp.minimum(g + 1, NG - 1) * GRP
        pltpu.async_copy(sc.at[pl.ds(gnxt, GRP)], sixn, isem)
        pltpu.async_copy(gc.at[pl.ds(gnxt, GRP)], gixn, isem)
        pltpu.async_copy(xc.at[gixq.at[0]], r0, gsa)
        for b in range(GRP):
          rx, gsx = (r0, gsa) if b % 2 == 0 else (r1, gsb)
          ry, gsy = (r1, gsb) if b % 2 == 0 else (r0, gsa)
          pltpu.make_async_copy(xc.at[gixq.at[b]], rx, gsx).wait()
          if b + 1 < GRP:
            pltpu.async_copy(xc.at[gixq.at[b + 1]], ry, gsy)
          pltpu.sync_copy(rx, acc.at[sixq.at[b]], add=True)
      return 0

    lax.fori_loop(0, NG // 2, pair, 0)
    pltpu.make_async_copy(sc.at[pl.ds(base, GRP)], six0, isem).wait()
    pltpu.make_async_copy(gc.at[pl.ds(base, GRP)], gix0, isem).wait()

  plsc.subcore_barrier()
  pltpu.sync_copy(acc.at[pl.ds(s * ROWS_PT, ROWS_PT)],
                  out.at[c].at[pl.ds(s * ROWS_PT, ROWS_PT)])


def _make_agg(with_gather):
  mesh = plsc.VectorSubcoreMesh(core_axis_name="c", subcore_axis_name="s")
  return functools.partial(
      pl.kernel,
      out_type=jax.ShapeDtypeStruct((2, NP, H), jnp.float32),
      mesh=mesh,
      scratch_types=[
          pltpu.VMEM((GRP, C), jnp.int32),
          pltpu.VMEM((GRP, C), jnp.int32),
          pltpu.VMEM((GRP, C), jnp.int32),
          pltpu.VMEM((GRP, C), jnp.int32),
          pltpu.VMEM((C, H), jnp.float32),
          pltpu.VMEM((C, H), jnp.float32),
          pltpu.VMEM_SHARED((NP, H), jnp.float32),
          pltpu.SemaphoreType.DMA,
          pltpu.SemaphoreType.DMA,
          pltpu.SemaphoreType.DMA,
          pltpu.SemaphoreType.DMA,
          pltpu.SemaphoreType.DMA,
      ],
  )(functools.partial(_agg_body, with_gather=with_gather))


_agg_sum = _make_agg(True)
_agg_cnt = _make_agg(False)

_DN = (((1,), (0,)), ((), ()))


def _proj_body(xu_ref, xb_ref, w_ref, b_ref, out_ref):
  d = pl.program_id(0)
  x = jnp.where(d == 0, xu_ref[...], xb_ref[...])
  y = lax.dot_general(x, w_ref[0], _DN, precision=lax.Precision.HIGHEST)
  out_ref[0] = y + b_ref[0]


def _proj(xu, xb, wst, bst):
  return pl.pallas_call(
      _proj_body,
      grid=(2, NB),
      in_specs=[
          pl.BlockSpec((R, H), lambda d, i: (i, 0)),
          pl.BlockSpec((R, H), lambda d, i: (i, 0)),
          pl.BlockSpec((1, H, H), lambda d, i: (d, 0, 0)),
          pl.BlockSpec((1, 1, H), lambda d, i: (d, 0, 0)),
      ],
      out_specs=pl.BlockSpec((1, R, H), lambda d, i: (d, i, 0)),
      out_shape=jax.ShapeDtypeStruct((2, NP, H), jnp.float32),
  )(xu, xb, wst, bst)


def _dense_body(sum_ref, cnt_ref, x_ref, wl_ref, wr_ref, bl_ref, g_ref,
                b2_ref, out_ref, *, ln):
  sums = sum_ref[0]
  cnt = jnp.maximum(cnt_ref[0][:, :1], 1.0)
  agg = sums / cnt
  x = x_ref[0]
  y = lax.dot_general(agg, wl_ref[0], _DN, precision=lax.Precision.HIGHEST)
  y = y + lax.dot_general(x, wr_ref[0], _DN, precision=lax.Precision.HIGHEST)
  y = jnp.maximum(y + bl_ref[0], 0.0)
  if ln:
    m = jnp.mean(y, axis=1, keepdims=True)
    v = jnp.mean((y - m) * (y - m), axis=1, keepdims=True)
    y = (y - m) * lax.rsqrt(v + 1e-5) * g_ref[0] + b2_ref[0]
  out_ref[0] = y


def _dense(ln):
  return pl.pallas_call(
      functools.partial(_dense_body, ln=ln),
      grid=(2, NB),
      in_specs=[
          pl.BlockSpec((1, R, H), lambda d, i: (1 - d, i, 0)),
          pl.BlockSpec((1, R, H), lambda d, i: (1 - d, i, 0)),
          pl.BlockSpec((1, R, H), lambda d, i: (d, i, 0)),
          pl.BlockSpec((1, H, H), lambda d, i: (0, 0, 0)),
          pl.BlockSpec((1, H, H), lambda d, i: (0, 0, 0)),
          pl.BlockSpec((1, 1, H), lambda d, i: (0, 0, 0)),
          pl.BlockSpec((1, 1, H), lambda d, i: (d, 0, 0)),
          pl.BlockSpec((1, 1, H), lambda d, i: (d, 0, 0)),
      ],
      out_specs=pl.BlockSpec((1, R, H), lambda d, i: (d, i, 0)),
      out_shape=jax.ShapeDtypeStruct((2, NP, H), jnp.float32),
  )


_dense_mid = _dense(False)
_dense_last = _dense(True)


def _prep_idx(a):
  pad = jnp.full((EPAD - E,), DUMMY, jnp.int32)
  return jnp.concatenate([a.astype(jnp.int32), pad]).reshape(NCH, C)


def kernel(x_user, x_book, edge_index_ub, edge_index_bu, Wu, bu, Wb, bb,
           Wl0, Wr0, bl0, Wl1, Wr1, bl1, Wl2, Wr2, bl2, gu, betau, gb, betab):
  f32 = jnp.float32
  gidx = jnp.stack([_prep_idx(edge_index_ub[0]), _prep_idx(edge_index_bu[0])])
  sidx = jnp.stack([_prep_idx(edge_index_ub[1]), _prep_idx(edge_index_bu[1])])
  zrows = jnp.zeros((ROWS_PT, H), f32)
  orows = jnp.ones((C, H), f32)

  wst = jnp.stack([Wu, Wb])
  bst = jnp.stack([bu, bb])[:, None, :]
  g2 = jnp.stack([gu, gb])[:, None, :]
  beta2 = jnp.stack([betau, betab])[:, None, :]

  x2 = _proj(jnp.pad(x_user, ((0, NP - N), (0, 0))),
             jnp.pad(x_book, ((0, NP - N), (0, 0))), wst, bst)
  cnts = _agg_cnt(x2, gidx, sidx, zrows, orows)

  layers = [(Wl0, Wr0, bl0, False), (Wl1, Wr1, bl1, False),
            (Wl2, Wr2, bl2, True)]
  for (wl, wr, bl, last) in layers:
    sums = _agg_sum(x2, gidx, sidx, zrows, orows)
    dense = _dense_last if last else _dense_mid
    x2 = dense(sums, cnts, x2, wl[None], wr[None], bl[None, None], g2, beta2)

  return (x2[0, :N], x2[1, :N])


# speedup vs baseline: 3.9986x; 1.0337x over previous
import functools

import jax
import jax.numpy as jnp
from jax import lax
from jax.experimental import pallas as pl
from jax.experimental.pallas import tpu as pltpu
from jax.experimental.pallas import tpu_sc as plsc

N = 10000
E = 320000
H = 128
C = 128
NT = 16
CPT = 160
NCH = CPT * NT
EPAD = NCH * C
NP = 10240
DUMMY = 10008
ROWS_PT = NP // NT
R = 1024
NB = NP // R


GRP = 16
NG = CPT // GRP


def _agg_body(x2, gidx, sidx, zrows, out, six0, gix0, six1, gix1,
              r0, r1, acc, gsa, gsb, isem):
  c = lax.axis_index("c")
  s = lax.axis_index("s")
  base = s * CPT
  xc = x2.at[c]
  sc = sidx.at[c]
  gc = gidx.at[c]

  pltpu.async_copy(sc.at[pl.ds(base, GRP)], six0, isem)
  pltpu.async_copy(gc.at[pl.ds(base, GRP)], gix0, isem)
  pltpu.sync_copy(zrows, acc.at[pl.ds(s * ROWS_PT, ROWS_PT)])
  plsc.subcore_barrier()

  rings = ((six0, gix0), (six1, gix1))

  def pair(p, _):
    for qg in (0, 1):
      g = p * 2 + qg
      sixq, gixq = rings[qg]
      sixn, gixn = rings[1 - qg]
      pltpu.make_async_copy(sc.at[pl.ds(base, GRP)], sixq, isem).wait()
      pltpu.make_async_copy(gc.at[pl.ds(base, GRP)], gixq, isem).wait()
      gnxt = base + jnp.minimum(g + 1, NG - 1) * GRP
      pltpu.async_copy(sc.at[pl.ds(gnxt, GRP)], sixn, isem)
      pltpu.async_copy(gc.at[pl.ds(gnxt, GRP)], gixn, isem)
      pltpu.async_copy(xc.at[gixq.at[0]], r0, gsa)
      for b in range(GRP):
        rx, gsx = (r0, gsa) if b % 2 == 0 else (r1, gsb)
        ry, gsy = (r1, gsb) if b % 2 == 0 else (r0, gsa)
        pltpu.make_async_copy(xc.at[gixq.at[b]], rx, gsx).wait()
        if b + 1 < GRP:
          pltpu.async_copy(xc.at[gixq.at[b + 1]], ry, gsy)
        pltpu.sync_copy(rx, acc.at[sixq.at[b]], add=True)
    return 0

  lax.fori_loop(0, NG // 2, pair, 0)
  pltpu.make_async_copy(sc.at[pl.ds(base, GRP)], six0, isem).wait()
  pltpu.make_async_copy(gc.at[pl.ds(base, GRP)], gix0, isem).wait()

  plsc.subcore_barrier()
  pltpu.sync_copy(acc.at[pl.ds(s * ROWS_PT, ROWS_PT)],
                  out.at[c].at[pl.ds(s * ROWS_PT, ROWS_PT)])


_agg_sum = functools.partial(
    pl.kernel,
    out_type=jax.ShapeDtypeStruct((2, NP, H), jnp.float32),
    mesh=plsc.VectorSubcoreMesh(core_axis_name="c", subcore_axis_name="s"),
    scratch_types=[
        pltpu.VMEM((GRP, C), jnp.int32),
        pltpu.VMEM((GRP, C), jnp.int32),
        pltpu.VMEM((GRP, C), jnp.int32),
        pltpu.VMEM((GRP, C), jnp.int32),
        pltpu.VMEM((C, H), jnp.float32),
        pltpu.VMEM((C, H), jnp.float32),
        pltpu.VMEM_SHARED((NP, H), jnp.float32),
        pltpu.SemaphoreType.DMA,
        pltpu.SemaphoreType.DMA,
        pltpu.SemaphoreType.DMA,
    ],
)(_agg_body)


def _cnt_body(sidx, zrows, orows, out, sixr, r0, acc):
  c = lax.axis_index("c")
  s = lax.axis_index("s")
  base = s * CPT
  pltpu.sync_copy(orows, r0)
  pltpu.sync_copy(zrows, acc.at[pl.ds(s * ROWS_PT, ROWS_PT)])
  plsc.subcore_barrier()

  def group(g, _):
    pltpu.sync_copy(sidx.at[c].at[pl.ds(base + g * GRP, GRP)], sixr)
    for b in range(GRP):
      pltpu.sync_copy(r0, acc.at[sixr.at[b]], add=True)
    return 0

  lax.fori_loop(0, NG, group, 0)
  plsc.subcore_barrier()
  pltpu.sync_copy(acc.at[pl.ds(s * ROWS_PT, ROWS_PT)],
                  out.at[c].at[pl.ds(s * ROWS_PT, ROWS_PT)])


_cnt = functools.partial(
    pl.kernel,
    out_type=jax.ShapeDtypeStruct((2, NP, H), jnp.float32),
    mesh=plsc.VectorSubcoreMesh(core_axis_name="c", subcore_axis_name="s"),
    scratch_types=[
        pltpu.VMEM((GRP, C), jnp.int32),
        pltpu.VMEM((C, H), jnp.float32),
        pltpu.VMEM_SHARED((NP, H), jnp.float32),
    ],
)(_cnt_body)

_DN = (((1,), (0,)), ((), ()))


def _proj_body(xu_ref, xb_ref, w_ref, b_ref, out_ref):
  d = pl.program_id(0)
  x = jnp.where(d == 0, xu_ref[...], xb_ref[...])
  y = lax.dot_general(x, w_ref[0], _DN, precision=lax.Precision.HIGHEST)
  out_ref[0] = y + b_ref[0]


def _proj(xu, xb, wst, bst):
  return pl.pallas_call(
      _proj_body,
      grid=(2, NB),
      in_specs=[
          pl.BlockSpec((R, H), lambda d, i: (i, 0)),
          pl.BlockSpec((R, H), lambda d, i: (i, 0)),
          pl.BlockSpec((1, H, H), lambda d, i: (d, 0, 0)),
          pl.BlockSpec((1, 1, H), lambda d, i: (d, 0, 0)),
      ],
      out_specs=pl.BlockSpec((1, R, H), lambda d, i: (d, i, 0)),
      out_shape=jax.ShapeDtypeStruct((2, NP, H), jnp.float32),
  )(xu, xb, wst, bst)


def _dense_body(sum_ref, cnt_ref, x_ref, wl_ref, wr_ref, bl_ref, g_ref,
                b2_ref, out_ref, *, ln):
  sums = sum_ref[0]
  cnt = jnp.maximum(cnt_ref[0][:, :1], 1.0)
  agg = sums / cnt
  x = x_ref[0]
  y = lax.dot_general(agg, wl_ref[0], _DN, precision=lax.Precision.HIGHEST)
  y = y + lax.dot_general(x, wr_ref[0], _DN, precision=lax.Precision.HIGHEST)
  y = jnp.maximum(y + bl_ref[0], 0.0)
  if ln:
    m = jnp.mean(y, axis=1, keepdims=True)
    v = jnp.mean((y - m) * (y - m), axis=1, keepdims=True)
    y = (y - m) * lax.rsqrt(v + 1e-5) * g_ref[0] + b2_ref[0]
  out_ref[0] = y


def _dense(ln):
  return pl.pallas_call(
      functools.partial(_dense_body, ln=ln),
      grid=(2, NB),
      in_specs=[
          pl.BlockSpec((1, R, H), lambda d, i: (1 - d, i, 0)),
          pl.BlockSpec((1, R, H), lambda d, i: (1 - d, i, 0)),
          pl.BlockSpec((1, R, H), lambda d, i: (d, i, 0)),
          pl.BlockSpec((1, H, H), lambda d, i: (0, 0, 0)),
          pl.BlockSpec((1, H, H), lambda d, i: (0, 0, 0)),
          pl.BlockSpec((1, 1, H), lambda d, i: (0, 0, 0)),
          pl.BlockSpec((1, 1, H), lambda d, i: (d, 0, 0)),
          pl.BlockSpec((1, 1, H), lambda d, i: (d, 0, 0)),
      ],
      out_specs=pl.BlockSpec((1, R, H), lambda d, i: (d, i, 0)),
      out_shape=jax.ShapeDtypeStruct((2, NP, H), jnp.float32),
  )


_dense_mid = _dense(False)
_dense_last = _dense(True)


def _prep_idx(a):
  pad = jnp.full((EPAD - E,), DUMMY, jnp.int32)
  return jnp.concatenate([a.astype(jnp.int32), pad]).reshape(NCH, C)


def kernel(x_user, x_book, edge_index_ub, edge_index_bu, Wu, bu, Wb, bb,
           Wl0, Wr0, bl0, Wl1, Wr1, bl1, Wl2, Wr2, bl2, gu, betau, gb, betab):
  f32 = jnp.float32
  gidx = jnp.stack([_prep_idx(edge_index_ub[0]), _prep_idx(edge_index_bu[0])])
  sidx = jnp.stack([_prep_idx(edge_index_ub[1]), _prep_idx(edge_index_bu[1])])
  zrows = jnp.zeros((ROWS_PT, H), f32)
  orows = jnp.ones((C, H), f32)

  wst = jnp.stack([Wu, Wb])
  bst = jnp.stack([bu, bb])[:, None, :]
  g2 = jnp.stack([gu, gb])[:, None, :]
  beta2 = jnp.stack([betau, betab])[:, None, :]

  x2 = _proj(jnp.pad(x_user, ((0, NP - N), (0, 0))),
             jnp.pad(x_book, ((0, NP - N), (0, 0))), wst, bst)
  cnts = _cnt(sidx, zrows, orows)

  layers = [(Wl0, Wr0, bl0, False), (Wl1, Wr1, bl1, False),
            (Wl2, Wr2, bl2, True)]
  for (wl, wr, bl, last) in layers:
    sums = _agg_sum(x2, gidx, sidx, zrows)
    dense = _dense_last if last else _dense_mid
    x2 = dense(sums, cnts, x2, wl[None], wr[None], bl[None, None], g2, beta2)

  return (x2[0, :N], x2[1, :N])
